# Initial kernel scaffold; baseline (speedup 1.0000x reference)
import jax
import jax.numpy as jnp
from jax.experimental import pallas as pl


def kernel(x, edge_index, W1, a_src1, a_dst1, b1, W2, a_src2, a_dst2, b2, W3, a_src3, a_dst3, b3):
    raise NotImplementedError("write your pallas kernel here")


# speedup vs baseline: 10.9653x; 10.9653x over previous
import functools

import jax
import jax.numpy as jnp
from jax import lax
from jax.experimental import pallas as pl
from jax.experimental.pallas import tpu as pltpu
from jax.experimental.pallas import tpu_sc as plsc

N = 10000
NP = 10240
E = 320000
NT = 16
RPT = NP // NT
DB = 400
MB = 400
F32 = jnp.float32
I32 = jnp.int32


def _mesh():
    return plsc.VectorSubcoreMesh(core_axis_name="c", subcore_axis_name="s",
                                  num_cores=2, num_subcores=NT)


def _running_cmax(i, a_s, a_d, m_ref, c_ref):
    ms = jnp.max(a_s)
    md = jnp.max(a_d)
    ms = jnp.maximum(jnp.where(i == 0, -jnp.inf, m_ref[0]), ms)
    md = jnp.maximum(jnp.where(i == 0, -jnp.inf, m_ref[1]), md)
    m_ref[0] = ms
    m_ref[1] = md
    tot = ms + md
    c = jnp.where(tot >= 0.0, tot, 0.2 * tot)
    c_ref[...] = jnp.full((1, 128), c, F32)


def _tc_head(x, wcm, ascm, adcm):
    n, din = x.shape
    nch, _, cw = wcm.shape
    br = 1024
    grid = (n // br,)

    def kern(x_ref, w_ref, as_ref, ad_ref, y_ref, ao_ref, bo_ref, c_ref, m_ref):
        i = pl.program_id(0)
        xb = x_ref[...]
        a_s = jnp.zeros((br, 16), F32)
        a_d = jnp.zeros((br, 16), F32)
        for c in range(nch):
            yc = jnp.dot(xb, w_ref[c], preferred_element_type=F32)
            y_ref[c] = yc
            a_s = a_s + jnp.dot(yc, as_ref[c], preferred_element_type=F32)
            a_d = a_d + jnp.dot(yc, ad_ref[c], preferred_element_type=F32)
        ao_ref[...] = a_s
        bo_ref[...] = a_d
        _running_cmax(i, a_s, a_d, m_ref, c_ref)

    return pl.pallas_call(
        kern,
        grid=grid,
        in_specs=[
            pl.BlockSpec((br, din), lambda i: (i, 0)),
            pl.BlockSpec((nch, din, cw), lambda i: (0, 0, 0)),
            pl.BlockSpec((nch, cw, 16), lambda i: (0, 0, 0)),
            pl.BlockSpec((nch, cw, 16), lambda i: (0, 0, 0)),
        ],
        out_specs=[
            pl.BlockSpec((nch, br, cw), lambda i: (0, i, 0)),
            pl.BlockSpec((br, 16), lambda i: (i, 0)),
            pl.BlockSpec((br, 16), lambda i: (i, 0)),
            pl.BlockSpec((1, 128), lambda i: (0, 0)),
        ],
        out_shape=[
            jax.ShapeDtypeStruct((nch, n, cw), F32),
            jax.ShapeDtypeStruct((n, 16), F32),
            jax.ShapeDtypeStruct((n, 16), F32),
            jax.ShapeDtypeStruct((1, 128), F32),
        ],
        scratch_shapes=[pltpu.SMEM((2,), F32)],
    )(x, wcm, ascm, adcm)


def _tc_mid(acc_cm, den, bias_cm, cph_in, wcm, ascm, adcm):
    ncin, n, cwin = acc_cm.shape
    ncout, din, cwout = wcm.shape
    br = 1024
    grid = (n // br,)

    def kern(acc_ref, den_ref, b_ref, w_ref, as_ref, ad_ref,
             y_ref, ao_ref, bo_ref, c_ref, m_ref):
        i = pl.program_id(0)
        dsum = den_ref[0] + den_ref[1]
        dinv = 1.0 / (dsum + 1e-16)
        hs = []
        for c in range(ncin):
            hd = c // cph_in
            dc = jnp.broadcast_to(dinv[:, hd:hd + 1], (br, cwin))
            hc = acc_ref[c] * dc + b_ref[c]
            hs.append(jnp.where(hc > 0.0, hc, jnp.exp(hc) - 1.0))
        h = jnp.concatenate(hs, axis=1)
        a_s = jnp.zeros((br, 16), F32)
        a_d = jnp.zeros((br, 16), F32)
        for c in range(ncout):
            yc = jnp.dot(h, w_ref[c], preferred_element_type=F32)
            y_ref[c] = yc
            a_s = a_s + jnp.dot(yc, as_ref[c], preferred_element_type=F32)
            a_d = a_d + jnp.dot(yc, ad_ref[c], preferred_element_type=F32)
        ao_ref[...] = a_s
        bo_ref[...] = a_d
        _running_cmax(i, a_s, a_d, m_ref, c_ref)

    return pl.pallas_call(
        kern,
        grid=grid,
        in_specs=[
            pl.BlockSpec((ncin, br, cwin), lambda i: (0, i, 0)),
            pl.BlockSpec((2, br, 16), lambda i: (0, i, 0)),
            pl.BlockSpec((ncin, 1, cwin), lambda i: (0, 0, 0)),
            pl.BlockSpec((ncout, din, cwout), lambda i: (0, 0, 0)),
            pl.BlockSpec((ncout, cwout, 16), lambda i: (0, 0, 0)),
            pl.BlockSpec((ncout, cwout, 16), lambda i: (0, 0, 0)),
        ],
        out_specs=[
            pl.BlockSpec((ncout, br, cwout), lambda i: (0, i, 0)),
            pl.BlockSpec((br, 16), lambda i: (i, 0)),
            pl.BlockSpec((br, 16), lambda i: (i, 0)),
            pl.BlockSpec((1, 128), lambda i: (0, 0)),
        ],
        out_shape=[
            jax.ShapeDtypeStruct((ncout, n, cwout), F32),
            jax.ShapeDtypeStruct((n, 16), F32),
            jax.ShapeDtypeStruct((n, 16), F32),
            jax.ShapeDtypeStruct((1, 128), F32),
        ],
        scratch_shapes=[pltpu.SMEM((2,), F32)],
    )(acc_cm, den, bias_cm, wcm, ascm, adcm)


def _tc_final(acc_cm, den, bias_row):
    _, n, _ = acc_cm.shape
    br = 1024
    grid = (n // br,)

    def kern(acc_ref, den_ref, b_ref, o_ref):
        dsum = den_ref[0] + den_ref[1]
        dinv = 1.0 / (dsum + 1e-16)
        d0 = jnp.broadcast_to(dinv[:, 0:1], (br, 16))
        z = acc_ref[0] * d0 + b_ref[...]
        m = jnp.max(z, axis=1, keepdims=True)
        zs = z - m
        lse = jnp.log(jnp.sum(jnp.exp(zs), axis=1, keepdims=True))
        o_ref[...] = zs - lse

    return pl.pallas_call(
        kern,
        grid=grid,
        in_specs=[
            pl.BlockSpec((1, br, 16), lambda i: (0, i, 0)),
            pl.BlockSpec((2, br, 16), lambda i: (0, i, 0)),
            pl.BlockSpec((1, 16), lambda i: (0, 0)),
        ],
        out_specs=pl.BlockSpec((br, 16), lambda i: (i, 0)),
        out_shape=jax.ShapeDtypeStruct((n, 16), F32),
    )(acc_cm, den, bias_row)


def _tc_wrep(w, sel):
    nh = sel.shape[0]
    br = 4000
    grid = (E // br,)

    def kern(w_ref, s_ref, o_ref):
        wb = w_ref[...]
        for h in range(nh):
            o_ref[h] = jnp.dot(wb, s_ref[h], preferred_element_type=F32)

    return pl.pallas_call(
        kern,
        grid=grid,
        in_specs=[
            pl.BlockSpec((br, 16), lambda i: (i, 0)),
            pl.BlockSpec((nh, 16, 16), lambda i: (0, 0, 0)),
        ],
        out_specs=pl.BlockSpec((nh, br, 16), lambda i: (0, i, 0)),
        out_shape=jax.ShapeDtypeStruct((nh, E, 16), F32),
    )(w, sel)


def _sc_denom(a_s, a_d, cmax, src, dst, zeros):
    out_type = (
        jax.ShapeDtypeStruct((2, NP, 16), F32),
        jax.ShapeDtypeStruct((E, 16), F32),
    )
    scratch = [
        pltpu.VMEM((DB,), I32),
        pltpu.VMEM((DB,), I32),
        pltpu.VMEM((DB, 16), F32),
        pltpu.VMEM((DB, 16), F32),
        pltpu.VMEM((DB, 16), F32),
        pltpu.VMEM((16,), F32),
        pltpu.VMEM_SHARED((NP, 16), F32),
        pltpu.VMEM_SHARED((NP, 16), F32),
        pltpu.VMEM_SHARED((NP, 16), F32),
        pltpu.SemaphoreType.DMA,
    ]

    @functools.partial(pl.kernel, out_type=out_type, mesh=_mesh(),
                       scratch_types=scratch,
                       compiler_params=pltpu.CompilerParams(
                           use_tc_tiling_on_sc=False))
    def body(as_h, ad_h, c_h, src_h, dst_h, z_h, den_o, w_o,
             sidx, didx, sbuf, dbuf, ebuf, cvv, st, dt, dens, sem):
        cid = lax.axis_index("c")
        sid = lax.axis_index("s")
        wid = cid * NT + sid
        r0 = sid * RPT

        pltpu.sync_copy(c_h.at[0, pl.ds(0, 16)], cvv)
        cv = cvv[...]
        pltpu.sync_copy(z_h.at[pl.ds(r0, RPT)],
                        dens.at[pl.ds(r0, RPT)])
        pltpu.sync_copy(as_h.at[pl.ds(r0, RPT)], st.at[pl.ds(r0, RPT)])
        pltpu.sync_copy(ad_h.at[pl.ds(r0, RPT)], dt.at[pl.ds(r0, RPT)])
        plsc.subcore_barrier()

        def batch(b, _):
            base = wid * (E // 32) + b * DB
            pltpu.sync_copy(src_h.at[pl.ds(base, DB)], sidx)
            pltpu.sync_copy(dst_h.at[pl.ds(base, DB)], didx)
            pltpu.async_copy(st.at[sidx], sbuf, sem).wait()
            pltpu.async_copy(dt.at[didx], dbuf, sem).wait()

            def edge(e, _):
                l16 = sbuf[e] + dbuf[e]
                l16 = jnp.where(l16 >= 0.0, l16, l16 * 0.2)
                ebuf[e] = jnp.exp(l16 - cv)
                return 0
            lax.fori_loop(0, DB, edge, 0)
            pltpu.sync_copy(ebuf, dens.at[didx], add=True)
            pltpu.sync_copy(ebuf, w_o.at[pl.ds(base, DB)])
            return 0
        lax.fori_loop(0, (E // 32) // DB, batch, 0)
        plsc.subcore_barrier()
        pltpu.sync_copy(dens.at[pl.ds(r0, RPT)],
                        den_o.at[cid, pl.ds(r0, RPT)])

    return body(a_s, a_d, cmax, src, dst, zeros)


def _sc_msg(y_cm, wrep, src, dst, zeros, chunk, nchunks, cph):
    half = (nchunks + 1) // 2
    nbat = (E // NT) // MB

    scratch = [
        pltpu.VMEM((MB,), I32),
        pltpu.VMEM((MB,), I32),
        pltpu.VMEM((MB, 16), F32),
        pltpu.VMEM((MB, chunk), F32),
        pltpu.VMEM_SHARED((NP, chunk), F32),
        pltpu.VMEM_SHARED((NP, chunk), F32),
        pltpu.SemaphoreType.DMA,
    ]
    nv = chunk // 16

    @functools.partial(pl.kernel,
                       out_type=jax.ShapeDtypeStruct((nchunks, NP, chunk), F32),
                       mesh=_mesh(), scratch_types=scratch,
                       compiler_params=pltpu.CompilerParams(
                           use_tc_tiling_on_sc=False))
    def body(y_h, w_h, src_h, dst_h, z_h, acc_o,
             sidx, didx, wrbuf, rows, tables, accs, sem):
        cid = lax.axis_index("c")
        sid = lax.axis_index("s")
        r0 = sid * RPT

        def do_chunk(j, _):
            gc = cid * half + j

            @pl.when(gc < nchunks)
            def _():
                h = gc // cph
                pltpu.sync_copy(y_h.at[gc, pl.ds(r0, RPT)],
                                tables.at[pl.ds(r0, RPT)])
                pltpu.sync_copy(z_h.at[pl.ds(r0, RPT)],
                                accs.at[pl.ds(r0, RPT)])
                plsc.subcore_barrier()

                def batch(b, _):
                    base = sid * (E // NT) + b * MB
                    pltpu.sync_copy(src_h.at[pl.ds(base, MB)], sidx)
                    pltpu.sync_copy(dst_h.at[pl.ds(base, MB)], didx)
                    pltpu.sync_copy(w_h.at[h, pl.ds(base, MB)], wrbuf)
                    pltpu.async_copy(tables.at[sidx], rows, sem).wait()

                    def edge(e, _):
                        wv = wrbuf[e]
                        for k in range(nv):
                            rows[e, pl.ds(k * 16, 16)] = (
                                rows[e, pl.ds(k * 16, 16)] * wv)
                        return 0
                    lax.fori_loop(0, MB, edge, 0)
                    pltpu.sync_copy(rows, accs.at[didx], add=True)
                    return 0
                lax.fori_loop(0, nbat, batch, 0)
                plsc.subcore_barrier()
                pltpu.sync_copy(accs.at[pl.ds(r0, RPT)],
                                acc_o.at[gc, pl.ds(r0, RPT)])
                plsc.subcore_barrier()
            return 0
        lax.fori_loop(0, half, do_chunk, 0)

    return body(y_cm, wrep, src, dst, zeros)


def _block_diag_att16(a, cw):
    heads, f = a.shape
    eye = jnp.eye(heads, dtype=F32)
    m = (eye[:, None, :] * a[:, :, None]).reshape(heads * f, heads)
    m = jnp.pad(m, ((0, 0), (0, 16 - heads)))
    return m.reshape(-1, cw, 16)


def kernel(x, edge_index, W1, a_src1, a_dst1, b1, W2, a_src2, a_dst2, b2,
           W3, a_src3, a_dst3, b3):
    src = edge_index[0]
    dst = edge_index[1]

    w1cm = W1.reshape(128, 16, 64).transpose(1, 0, 2)
    as1 = _block_diag_att16(a_src1, 64)
    ad1 = _block_diag_att16(a_dst1, 64)
    w2cm = W2.reshape(1024, 8, 64).transpose(1, 0, 2)
    as2 = _block_diag_att16(a_src2, 64)
    ad2 = _block_diag_att16(a_dst2, 64)
    w3cm = jnp.pad(W3, ((0, 0), (0, 13)))[None]
    as3 = jnp.zeros((1, 16, 16), F32).at[0, :3, 0].set(a_src3[0])
    ad3 = jnp.zeros((1, 16, 16), F32).at[0, :3, 0].set(a_dst3[0])
    sel8 = jnp.zeros((8, 16, 16), F32)
    sel8 = sel8.at[jnp.arange(8), jnp.arange(8), :].set(1.0)
    sel1 = sel8[:1]
    b1cm = b1.reshape(16, 1, 64)
    b2cm = b2.reshape(8, 1, 64)
    b3r = jnp.concatenate([b3, jnp.full((13,), -1e30, F32)]).reshape(1, 16)
    zeros16 = jnp.zeros((NP, 16), F32)
    zeros64 = jnp.zeros((NP, 64), F32)
    xp = jnp.pad(x, ((0, NP - N), (0, 0)))

    y1, s1, d1, c1 = _tc_head(xp, w1cm, as1, ad1)
    den1, w1e = _sc_denom(s1, d1, c1, src, dst, zeros16)
    wrep1 = _tc_wrep(w1e, sel8)
    acc1 = _sc_msg(y1, wrep1, src, dst, zeros64, 64, 16, 2)

    y2, s2, d2, c2 = _tc_mid(acc1, den1, b1cm, 2, w2cm, as2, ad2)
    den2, w2e = _sc_denom(s2, d2, c2, src, dst, zeros16)
    wrep2 = _tc_wrep(w2e, sel8)
    acc2 = _sc_msg(y2, wrep2, src, dst, zeros64, 64, 8, 1)

    y3, s3, d3, c3 = _tc_mid(acc2, den2, b2cm, 1, w3cm, as3, ad3)
    den3, w3e = _sc_denom(s3, d3, c3, src, dst, zeros16)
    wrep3 = _tc_wrep(w3e, sel1)
    acc3 = _sc_msg(y3, wrep3, src, dst, zeros16, 16, 1, 1)

    out16 = _tc_final(acc3, den3, b3r)
    return out16[:N, :3]


# speedup vs baseline: 11.1536x; 1.0172x over previous
import functools

import jax
import jax.numpy as jnp
from jax import lax
from jax.experimental import pallas as pl
from jax.experimental.pallas import tpu as pltpu
from jax.experimental.pallas import tpu_sc as plsc

N = 10000
NP = 10240
E = 320000
NT = 16
RPT = NP // NT
DB = 400
MB = 200
F32 = jnp.float32
I32 = jnp.int32


def _mesh():
    return plsc.VectorSubcoreMesh(core_axis_name="c", subcore_axis_name="s",
                                  num_cores=2, num_subcores=NT)


def _running_cmax(i, a_s, a_d, m_ref, c_ref):
    ms = jnp.max(a_s)
    md = jnp.max(a_d)
    ms = jnp.maximum(jnp.where(i == 0, -jnp.inf, m_ref[0]), ms)
    md = jnp.maximum(jnp.where(i == 0, -jnp.inf, m_ref[1]), md)
    m_ref[0] = ms
    m_ref[1] = md
    tot = ms + md
    c = jnp.where(tot >= 0.0, tot, 0.2 * tot)
    c_ref[...] = jnp.full((1, 128), c, F32)


def _tc_head(x, wcm, ascm, adcm):
    n, din = x.shape
    nch, _, cw = wcm.shape
    br = 1024
    grid = (n // br,)

    def kern(x_ref, w_ref, as_ref, ad_ref, y_ref, ao_ref, bo_ref, c_ref, m_ref):
        i = pl.program_id(0)
        xb = x_ref[...]
        a_s = jnp.zeros((br, 16), F32)
        a_d = jnp.zeros((br, 16), F32)
        for c in range(nch):
            yc = jnp.dot(xb, w_ref[c], preferred_element_type=F32)
            y_ref[c] = yc
            a_s = a_s + jnp.dot(yc, as_ref[c], preferred_element_type=F32)
            a_d = a_d + jnp.dot(yc, ad_ref[c], preferred_element_type=F32)
        ao_ref[...] = a_s
        bo_ref[...] = a_d
        _running_cmax(i, a_s, a_d, m_ref, c_ref)

    return pl.pallas_call(
        kern,
        grid=grid,
        in_specs=[
            pl.BlockSpec((br, din), lambda i: (i, 0)),
            pl.BlockSpec((nch, din, cw), lambda i: (0, 0, 0)),
            pl.BlockSpec((nch, cw, 16), lambda i: (0, 0, 0)),
            pl.BlockSpec((nch, cw, 16), lambda i: (0, 0, 0)),
        ],
        out_specs=[
            pl.BlockSpec((nch, br, cw), lambda i: (0, i, 0)),
            pl.BlockSpec((br, 16), lambda i: (i, 0)),
            pl.BlockSpec((br, 16), lambda i: (i, 0)),
            pl.BlockSpec((1, 128), lambda i: (0, 0)),
        ],
        out_shape=[
            jax.ShapeDtypeStruct((nch, n, cw), F32),
            jax.ShapeDtypeStruct((n, 16), F32),
            jax.ShapeDtypeStruct((n, 16), F32),
            jax.ShapeDtypeStruct((1, 128), F32),
        ],
        scratch_shapes=[pltpu.SMEM((2,), F32)],
    )(x, wcm, ascm, adcm)


def _tc_mid(acc_cm, den, bias_cm, cph_in, wcm, ascm, adcm):
    ncin, n, cwin = acc_cm.shape
    ncout, din, cwout = wcm.shape
    br = 1024
    grid = (n // br,)

    def kern(acc_ref, den_ref, b_ref, w_ref, as_ref, ad_ref,
             y_ref, ao_ref, bo_ref, c_ref, m_ref):
        i = pl.program_id(0)
        dsum = den_ref[0] + den_ref[1]
        dinv = 1.0 / (dsum + 1e-16)
        hs = []
        for c in range(ncin):
            hd = c // cph_in
            dc = jnp.broadcast_to(dinv[:, hd:hd + 1], (br, cwin))
            hc = acc_ref[c] * dc + b_ref[c]
            hs.append(jnp.where(hc > 0.0, hc, jnp.exp(hc) - 1.0))
        h = jnp.concatenate(hs, axis=1)
        a_s = jnp.zeros((br, 16), F32)
        a_d = jnp.zeros((br, 16), F32)
        for c in range(ncout):
            yc = jnp.dot(h, w_ref[c], preferred_element_type=F32)
            y_ref[c] = yc
            a_s = a_s + jnp.dot(yc, as_ref[c], preferred_element_type=F32)
            a_d = a_d + jnp.dot(yc, ad_ref[c], preferred_element_type=F32)
        ao_ref[...] = a_s
        bo_ref[...] = a_d
        _running_cmax(i, a_s, a_d, m_ref, c_ref)

    return pl.pallas_call(
        kern,
        grid=grid,
        in_specs=[
            pl.BlockSpec((ncin, br, cwin), lambda i: (0, i, 0)),
            pl.BlockSpec((2, br, 16), lambda i: (0, i, 0)),
            pl.BlockSpec((ncin, 1, cwin), lambda i: (0, 0, 0)),
            pl.BlockSpec((ncout, din, cwout), lambda i: (0, 0, 0)),
            pl.BlockSpec((ncout, cwout, 16), lambda i: (0, 0, 0)),
            pl.BlockSpec((ncout, cwout, 16), lambda i: (0, 0, 0)),
        ],
        out_specs=[
            pl.BlockSpec((ncout, br, cwout), lambda i: (0, i, 0)),
            pl.BlockSpec((br, 16), lambda i: (i, 0)),
            pl.BlockSpec((br, 16), lambda i: (i, 0)),
            pl.BlockSpec((1, 128), lambda i: (0, 0)),
        ],
        out_shape=[
            jax.ShapeDtypeStruct((ncout, n, cwout), F32),
            jax.ShapeDtypeStruct((n, 16), F32),
            jax.ShapeDtypeStruct((n, 16), F32),
            jax.ShapeDtypeStruct((1, 128), F32),
        ],
        scratch_shapes=[pltpu.SMEM((2,), F32)],
    )(acc_cm, den, bias_cm, wcm, ascm, adcm)


def _tc_final(acc_cm, den, bias_row):
    _, n, _ = acc_cm.shape
    br = 1024
    grid = (n // br,)

    def kern(acc_ref, den_ref, b_ref, o_ref):
        dsum = den_ref[0] + den_ref[1]
        dinv = 1.0 / (dsum + 1e-16)
        d0 = jnp.broadcast_to(dinv[:, 0:1], (br, 16))
        z = acc_ref[0] * d0 + b_ref[...]
        m = jnp.max(z, axis=1, keepdims=True)
        zs = z - m
        lse = jnp.log(jnp.sum(jnp.exp(zs), axis=1, keepdims=True))
        o_ref[...] = zs - lse

    return pl.pallas_call(
        kern,
        grid=grid,
        in_specs=[
            pl.BlockSpec((1, br, 16), lambda i: (0, i, 0)),
            pl.BlockSpec((2, br, 16), lambda i: (0, i, 0)),
            pl.BlockSpec((1, 16), lambda i: (0, 0)),
        ],
        out_specs=pl.BlockSpec((br, 16), lambda i: (i, 0)),
        out_shape=jax.ShapeDtypeStruct((n, 16), F32),
    )(acc_cm, den, bias_row)


def _tc_wrep(w, sel):
    nh = sel.shape[0]
    br = 4000
    grid = (E // br,)

    def kern(w_ref, s_ref, o_ref):
        wb = w_ref[...]
        for h in range(nh):
            o_ref[h] = jnp.dot(wb, s_ref[h], preferred_element_type=F32)

    return pl.pallas_call(
        kern,
        grid=grid,
        in_specs=[
            pl.BlockSpec((br, 16), lambda i: (i, 0)),
            pl.BlockSpec((nh, 16, 16), lambda i: (0, 0, 0)),
        ],
        out_specs=pl.BlockSpec((nh, br, 16), lambda i: (0, i, 0)),
        out_shape=jax.ShapeDtypeStruct((nh, E, 16), F32),
    )(w, sel)


def _sc_denom(a_s, a_d, cmax, src, dst, zeros):
    out_type = (
        jax.ShapeDtypeStruct((2, NP, 16), F32),
        jax.ShapeDtypeStruct((E, 16), F32),
    )
    scratch = [
        pltpu.VMEM((DB,), I32),
        pltpu.VMEM((DB,), I32),
        pltpu.VMEM((DB, 16), F32),
        pltpu.VMEM((DB, 16), F32),
        pltpu.VMEM((DB, 16), F32),
        pltpu.VMEM((16,), F32),
        pltpu.VMEM_SHARED((NP, 16), F32),
        pltpu.SemaphoreType.DMA,
    ]

    @functools.partial(pl.kernel, out_type=out_type, mesh=_mesh(),
                       scratch_types=scratch,
                       compiler_params=pltpu.CompilerParams(
                           use_tc_tiling_on_sc=False))
    def body(as_h, ad_h, c_h, src_h, dst_h, z_h, den_o, w_o,
             sidx, didx, sbuf, dbuf, ebuf, cvv, dens, sem):
        cid = lax.axis_index("c")
        sid = lax.axis_index("s")
        wid = cid * NT + sid
        r0 = sid * RPT

        pltpu.sync_copy(c_h.at[0, pl.ds(0, 16)], cvv)
        cv = cvv[...]
        pltpu.sync_copy(z_h.at[pl.ds(r0, RPT)],
                        dens.at[pl.ds(r0, RPT)])
        plsc.subcore_barrier()

        def batch(b, _):
            base = wid * (E // 32) + b * DB
            pltpu.sync_copy(src_h.at[pl.ds(base, DB)], sidx)
            pltpu.sync_copy(dst_h.at[pl.ds(base, DB)], didx)
            pltpu.async_copy(as_h.at[sidx], sbuf, sem).wait()
            pltpu.async_copy(ad_h.at[didx], dbuf, sem).wait()

            def edge(e, _):
                l16 = sbuf[e] + dbuf[e]
                l16 = jnp.where(l16 >= 0.0, l16, l16 * 0.2)
                ebuf[e] = jnp.exp(l16 - cv)
                return 0
            lax.fori_loop(0, DB, edge, 0)
            pltpu.sync_copy(ebuf, dens.at[didx], add=True)
            pltpu.sync_copy(ebuf, w_o.at[pl.ds(base, DB)])
            return 0
        lax.fori_loop(0, (E // 32) // DB, batch, 0)
        plsc.subcore_barrier()
        pltpu.sync_copy(dens.at[pl.ds(r0, RPT)],
                        den_o.at[cid, pl.ds(r0, RPT)])

    return body(a_s, a_d, cmax, src, dst, zeros)


def _sc_msg(y_cm, wrep, src, dst, zeros, chunk, nchunks, cph):
    half = (nchunks + 1) // 2
    nbat = (E // NT) // MB
    npair = nbat // 2

    scratch = [
        pltpu.VMEM((MB,), I32), pltpu.VMEM((MB,), I32),
        pltpu.VMEM((MB,), I32), pltpu.VMEM((MB,), I32),
        pltpu.VMEM((MB, 16), F32), pltpu.VMEM((MB, 16), F32),
        pltpu.VMEM((MB, chunk), F32), pltpu.VMEM((MB, chunk), F32),
        pltpu.VMEM_SHARED((NP, chunk), F32),
        pltpu.VMEM_SHARED((NP, chunk), F32),
        pltpu.SemaphoreType.DMA, pltpu.SemaphoreType.DMA,
        pltpu.SemaphoreType.DMA, pltpu.SemaphoreType.DMA,
    ]
    nv = chunk // 16

    @functools.partial(pl.kernel,
                       out_type=jax.ShapeDtypeStruct((nchunks, NP, chunk), F32),
                       mesh=_mesh(), scratch_types=scratch,
                       compiler_params=pltpu.CompilerParams(
                           use_tc_tiling_on_sc=False))
    def body(y_h, w_h, src_h, dst_h, z_h, acc_o,
             sidx0, sidx1, didx0, didx1, wr0, wr1, rows0, rows1,
             tables, accs, gsem0, gsem1, ssem0, ssem1):
        cid = lax.axis_index("c")
        sid = lax.axis_index("s")
        r0 = sid * RPT
        ebase = sid * (E // NT)
        bufs = ((sidx0, didx0, wr0, rows0, gsem0, ssem0),
                (sidx1, didx1, wr1, rows1, gsem1, ssem1))

        def fetch(h, b, buf):
            si, di, wr, rw, gs, _ = buf
            base = ebase + b * MB
            pltpu.sync_copy(src_h.at[pl.ds(base, MB)], si)
            pltpu.sync_copy(dst_h.at[pl.ds(base, MB)], di)
            pltpu.sync_copy(w_h.at[h, pl.ds(base, MB)], wr)
            pltpu.async_copy(tables.at[si], rw, gs)

        def compute(buf):
            si, di, wr, rw, gs, ss = buf
            pltpu.make_async_copy(tables.at[si], rw, gs).wait()

            def edge(e2, _):
                for u in range(2):
                    e = e2 * 2 + u
                    wv = wr[e]
                    for k in range(nv):
                        rw[e, pl.ds(k * 16, 16)] = (
                            rw[e, pl.ds(k * 16, 16)] * wv)
                return 0
            lax.fori_loop(0, MB // 2, edge, 0)
            pltpu.async_copy(rw, accs.at[di], ss, add=True)

        def swait(buf):
            _, di, _, rw, _, ss = buf
            pltpu.make_async_copy(rw, accs.at[di], ss).wait()

        def do_chunk(j, _):
            gc = cid * half + j

            @pl.when(gc < nchunks)
            def _():
                h = gc // cph
                pltpu.sync_copy(y_h.at[gc, pl.ds(r0, RPT)],
                                tables.at[pl.ds(r0, RPT)])
                pltpu.sync_copy(z_h.at[pl.ds(r0, RPT)],
                                accs.at[pl.ds(r0, RPT)])
                plsc.subcore_barrier()
                fetch(h, 0, bufs[0])

                def pair(p, _):
                    @pl.when(p > 0)
                    def _():
                        swait(bufs[1])
                    fetch(h, 2 * p + 1, bufs[1])
                    compute(bufs[0])

                    @pl.when(p < npair - 1)
                    def _():
                        swait(bufs[0])
                        fetch(h, 2 * p + 2, bufs[0])
                    compute(bufs[1])
                    return 0
                lax.fori_loop(0, npair, pair, 0)
                swait(bufs[0])
                swait(bufs[1])
                plsc.subcore_barrier()
                pltpu.sync_copy(accs.at[pl.ds(r0, RPT)],
                                acc_o.at[gc, pl.ds(r0, RPT)])
                plsc.subcore_barrier()
            return 0
        lax.fori_loop(0, half, do_chunk, 0)

    return body(y_cm, wrep, src, dst, zeros)


def _block_diag_att16(a, cw):
    heads, f = a.shape
    eye = jnp.eye(heads, dtype=F32)
    m = (eye[:, None, :] * a[:, :, None]).reshape(heads * f, heads)
    m = jnp.pad(m, ((0, 0), (0, 16 - heads)))
    return m.reshape(-1, cw, 16)


def kernel(x, edge_index, W1, a_src1, a_dst1, b1, W2, a_src2, a_dst2, b2,
           W3, a_src3, a_dst3, b3):
    src = edge_index[0]
    dst = edge_index[1]

    w1cm = W1.reshape(128, 16, 64).transpose(1, 0, 2)
    as1 = _block_diag_att16(a_src1, 64)
    ad1 = _block_diag_att16(a_dst1, 64)
    w2cm = W2.reshape(1024, 8, 64).transpose(1, 0, 2)
    as2 = _block_diag_att16(a_src2, 64)
    ad2 = _block_diag_att16(a_dst2, 64)
    w3cm = jnp.pad(W3, ((0, 0), (0, 13)))[None]
    as3 = jnp.zeros((1, 16, 16), F32).at[0, :3, 0].set(a_src3[0])
    ad3 = jnp.zeros((1, 16, 16), F32).at[0, :3, 0].set(a_dst3[0])
    sel8 = jnp.zeros((8, 16, 16), F32)
    sel8 = sel8.at[jnp.arange(8), jnp.arange(8), :].set(1.0)
    sel1 = sel8[:1]
    b1cm = b1.reshape(16, 1, 64)
    b2cm = b2.reshape(8, 1, 64)
    b3r = jnp.concatenate([b3, jnp.full((13,), -1e30, F32)]).reshape(1, 16)
    zeros16 = jnp.zeros((NP, 16), F32)
    zeros64 = jnp.zeros((NP, 64), F32)
    xp = jnp.pad(x, ((0, NP - N), (0, 0)))

    y1, s1, d1, c1 = _tc_head(xp, w1cm, as1, ad1)
    den1, w1e = _sc_denom(s1, d1, c1, src, dst, zeros16)
    wrep1 = _tc_wrep(w1e, sel8)
    acc1 = _sc_msg(y1, wrep1, src, dst, zeros64, 64, 16, 2)

    y2, s2, d2, c2 = _tc_mid(acc1, den1, b1cm, 2, w2cm, as2, ad2)
    den2, w2e = _sc_denom(s2, d2, c2, src, dst, zeros16)
    wrep2 = _tc_wrep(w2e, sel8)
    acc2 = _sc_msg(y2, wrep2, src, dst, zeros64, 64, 8, 1)

    y3, s3, d3, c3 = _tc_mid(acc2, den2, b2cm, 1, w3cm, as3, ad3)
    den3, w3e = _sc_denom(s3, d3, c3, src, dst, zeros16)
    wrep3 = _tc_wrep(w3e, sel1)
    acc3 = _sc_msg(y3, wrep3, src, dst, zeros16, 16, 1, 1)

    out16 = _tc_final(acc3, den3, b3r)
    return out16[:N, :3]


# speedup vs baseline: 17.3693x; 1.5573x over previous
import functools

import jax
import jax.numpy as jnp
from jax import lax
from jax.experimental import pallas as pl
from jax.experimental.pallas import tpu as pltpu
from jax.experimental.pallas import tpu_sc as plsc

N = 10000
NP = 10240
E = 320000
NT = 16
RPT = NP // NT
DB = 400
MB = 200
F32 = jnp.float32
I32 = jnp.int32


def _mesh():
    return plsc.VectorSubcoreMesh(core_axis_name="c", subcore_axis_name="s",
                                  num_cores=2, num_subcores=NT)


def _running_cmax(i, a_s, a_d, m_ref, c_ref):
    ms = jnp.max(a_s)
    md = jnp.max(a_d)
    ms = jnp.maximum(jnp.where(i == 0, -jnp.inf, m_ref[0]), ms)
    md = jnp.maximum(jnp.where(i == 0, -jnp.inf, m_ref[1]), md)
    m_ref[0] = ms
    m_ref[1] = md
    tot = ms + md
    c = jnp.where(tot >= 0.0, tot, 0.2 * tot)
    c_ref[...] = jnp.full((1, 128), c, F32)


def _tc_head(x, wcm, ascm, adcm):
    n, din = x.shape
    nch, _, cw = wcm.shape
    br = 1024
    grid = (n // br,)

    def kern(x_ref, w_ref, as_ref, ad_ref, y_ref, ao_ref, bo_ref, c_ref, m_ref):
        i = pl.program_id(0)
        xb = x_ref[...]
        a_s = jnp.zeros((br, 16), F32)
        a_d = jnp.zeros((br, 16), F32)
        for c in range(nch):
            yc = jnp.dot(xb, w_ref[c], preferred_element_type=F32)
            y_ref[c] = yc
            a_s = a_s + jnp.dot(yc, as_ref[c], preferred_element_type=F32)
            a_d = a_d + jnp.dot(yc, ad_ref[c], preferred_element_type=F32)
        ao_ref[...] = a_s
        bo_ref[...] = a_d
        _running_cmax(i, a_s, a_d, m_ref, c_ref)

    return pl.pallas_call(
        kern,
        grid=grid,
        in_specs=[
            pl.BlockSpec((br, din), lambda i: (i, 0)),
            pl.BlockSpec((nch, din, cw), lambda i: (0, 0, 0)),
            pl.BlockSpec((nch, cw, 16), lambda i: (0, 0, 0)),
            pl.BlockSpec((nch, cw, 16), lambda i: (0, 0, 0)),
        ],
        out_specs=[
            pl.BlockSpec((nch, br, cw), lambda i: (0, i, 0)),
            pl.BlockSpec((br, 16), lambda i: (i, 0)),
            pl.BlockSpec((br, 16), lambda i: (i, 0)),
            pl.BlockSpec((1, 128), lambda i: (0, 0)),
        ],
        out_shape=[
            jax.ShapeDtypeStruct((nch, n, cw), F32),
            jax.ShapeDtypeStruct((n, 16), F32),
            jax.ShapeDtypeStruct((n, 16), F32),
            jax.ShapeDtypeStruct((1, 128), F32),
        ],
        scratch_shapes=[pltpu.SMEM((2,), F32)],
    )(x, wcm, ascm, adcm)


def _tc_mid(acc_cm, den, bias_cm, wcm, ascm, adcm):
    ncin, n, cwin = acc_cm.shape
    ncout, din, cwout = wcm.shape
    br = 1024
    grid = (n // br,)

    def kern(acc_ref, den_ref, b_ref, w_ref, as_ref, ad_ref,
             y_ref, ao_ref, bo_ref, c_ref, m_ref):
        i = pl.program_id(0)
        dsum = den_ref[0] + den_ref[1]
        dinv = 1.0 / (dsum + 1e-16)
        dc = jnp.tile(dinv[:, :8], (1, cwin // 8))
        hs = []
        for c in range(ncin):
            hc = acc_ref[c] * dc + b_ref[c]
            hs.append(jnp.where(hc > 0.0, hc, jnp.exp(hc) - 1.0))
        h = jnp.concatenate(hs, axis=1)
        a_s = jnp.zeros((br, 16), F32)
        a_d = jnp.zeros((br, 16), F32)
        for c in range(ncout):
            yc = jnp.dot(h, w_ref[c], preferred_element_type=F32)
            y_ref[c] = yc
            a_s = a_s + jnp.dot(yc, as_ref[c], preferred_element_type=F32)
            a_d = a_d + jnp.dot(yc, ad_ref[c], preferred_element_type=F32)
        ao_ref[...] = a_s
        bo_ref[...] = a_d
        _running_cmax(i, a_s, a_d, m_ref, c_ref)

    return pl.pallas_call(
        kern,
        grid=grid,
        in_specs=[
            pl.BlockSpec((ncin, br, cwin), lambda i: (0, i, 0)),
            pl.BlockSpec((2, br, 16), lambda i: (0, i, 0)),
            pl.BlockSpec((ncin, 1, cwin), lambda i: (0, 0, 0)),
            pl.BlockSpec((ncout, din, cwout), lambda i: (0, 0, 0)),
            pl.BlockSpec((ncout, cwout, 16), lambda i: (0, 0, 0)),
            pl.BlockSpec((ncout, cwout, 16), lambda i: (0, 0, 0)),
        ],
        out_specs=[
            pl.BlockSpec((ncout, br, cwout), lambda i: (0, i, 0)),
            pl.BlockSpec((br, 16), lambda i: (i, 0)),
            pl.BlockSpec((br, 16), lambda i: (i, 0)),
            pl.BlockSpec((1, 128), lambda i: (0, 0)),
        ],
        out_shape=[
            jax.ShapeDtypeStruct((ncout, n, cwout), F32),
            jax.ShapeDtypeStruct((n, 16), F32),
            jax.ShapeDtypeStruct((n, 16), F32),
            jax.ShapeDtypeStruct((1, 128), F32),
        ],
        scratch_shapes=[pltpu.SMEM((2,), F32)],
    )(acc_cm, den, bias_cm, wcm, ascm, adcm)


def _tc_final(acc_cm, den, bias_row):
    _, n, _ = acc_cm.shape
    br = 1024
    grid = (n // br,)

    def kern(acc_ref, den_ref, b_ref, o_ref):
        dsum = den_ref[0] + den_ref[1]
        dinv = 1.0 / (dsum + 1e-16)
        d0 = jnp.broadcast_to(dinv[:, 0:1], (br, 16))
        z = acc_ref[0] * d0 + b_ref[...]
        m = jnp.max(z, axis=1, keepdims=True)
        zs = z - m
        lse = jnp.log(jnp.sum(jnp.exp(zs), axis=1, keepdims=True))
        o_ref[...] = zs - lse

    return pl.pallas_call(
        kern,
        grid=grid,
        in_specs=[
            pl.BlockSpec((1, br, 16), lambda i: (0, i, 0)),
            pl.BlockSpec((2, br, 16), lambda i: (0, i, 0)),
            pl.BlockSpec((1, 16), lambda i: (0, 0)),
        ],
        out_specs=pl.BlockSpec((br, 16), lambda i: (i, 0)),
        out_shape=jax.ShapeDtypeStruct((n, 16), F32),
    )(acc_cm, den, bias_row)


def _sc_denom(a_s, a_d, cmax, src, dst, zeros):
    out_type = (
        jax.ShapeDtypeStruct((2, NP, 16), F32),
        jax.ShapeDtypeStruct((E, 16), F32),
    )
    scratch = [
        pltpu.VMEM((DB,), I32),
        pltpu.VMEM((DB,), I32),
        pltpu.VMEM((DB, 16), F32),
        pltpu.VMEM((DB, 16), F32),
        pltpu.VMEM((DB, 16), F32),
        pltpu.VMEM((16,), F32),
        pltpu.VMEM_SHARED((NP, 16), F32),
        pltpu.SemaphoreType.DMA,
    ]

    @functools.partial(pl.kernel, out_type=out_type, mesh=_mesh(),
                       scratch_types=scratch,
                       compiler_params=pltpu.CompilerParams(
                           use_tc_tiling_on_sc=False))
    def body(as_h, ad_h, c_h, src_h, dst_h, z_h, den_o, w_o,
             sidx, didx, sbuf, dbuf, ebuf, cvv, dens, sem):
        cid = lax.axis_index("c")
        sid = lax.axis_index("s")
        wid = cid * NT + sid
        r0 = sid * RPT

        pltpu.sync_copy(c_h.at[0, pl.ds(0, 16)], cvv)
        cv = cvv[...]
        pltpu.sync_copy(z_h.at[pl.ds(r0, RPT)],
                        dens.at[pl.ds(r0, RPT)])
        plsc.subcore_barrier()

        def batch(b, _):
            base = wid * (E // 32) + b * DB
            pltpu.sync_copy(src_h.at[pl.ds(base, DB)], sidx)
            pltpu.sync_copy(dst_h.at[pl.ds(base, DB)], didx)
            pltpu.async_copy(as_h.at[sidx], sbuf, sem).wait()
            pltpu.async_copy(ad_h.at[didx], dbuf, sem).wait()

            def edge(e, _):
                l16 = sbuf[e] + dbuf[e]
                l16 = jnp.where(l16 >= 0.0, l16, l16 * 0.2)
                ebuf[e] = jnp.exp(l16 - cv)
                return 0
            lax.fori_loop(0, DB, edge, 0)
            pltpu.sync_copy(ebuf, dens.at[didx], add=True)
            pltpu.sync_copy(ebuf, w_o.at[pl.ds(base, DB)])
            return 0
        lax.fori_loop(0, (E // 32) // DB, batch, 0)
        plsc.subcore_barrier()
        pltpu.sync_copy(dens.at[pl.ds(r0, RPT)],
                        den_o.at[cid, pl.ds(r0, RPT)])

    return body(a_s, a_d, cmax, src, dst, zeros)


def _sc_msg(y_cm, w, src, dst, zeros, chunk, nchunks):
    half = (nchunks + 1) // 2
    nbat = (E // NT) // MB
    npair = nbat // 2

    scratch = [
        pltpu.VMEM((MB,), I32), pltpu.VMEM((MB,), I32),
        pltpu.VMEM((MB,), I32), pltpu.VMEM((MB,), I32),
        pltpu.VMEM((MB, 16), F32), pltpu.VMEM((MB, 16), F32),
        pltpu.VMEM((MB, chunk), F32), pltpu.VMEM((MB, chunk), F32),
        pltpu.VMEM_SHARED((NP, chunk), F32),
        pltpu.VMEM_SHARED((NP, chunk), F32),
        pltpu.SemaphoreType.DMA, pltpu.SemaphoreType.DMA,
        pltpu.SemaphoreType.DMA, pltpu.SemaphoreType.DMA,
    ]
    nv = chunk // 16

    @functools.partial(pl.kernel,
                       out_type=jax.ShapeDtypeStruct((nchunks, NP, chunk), F32),
                       mesh=_mesh(), scratch_types=scratch,
                       compiler_params=pltpu.CompilerParams(
                           use_tc_tiling_on_sc=False))
    def body(y_h, w_h, src_h, dst_h, z_h, acc_o,
             sidx0, sidx1, didx0, didx1, wr0, wr1, rows0, rows1,
             tables, accs, gsem0, gsem1, ssem0, ssem1):
        cid = lax.axis_index("c")
        sid = lax.axis_index("s")
        r0 = sid * RPT
        ebase = sid * (E // NT)
        bufs = ((sidx0, didx0, wr0, rows0, gsem0, ssem0),
                (sidx1, didx1, wr1, rows1, gsem1, ssem1))

        def fetch(b, buf):
            si, di, wr, rw, gs, _ = buf
            base = ebase + b * MB
            pltpu.sync_copy(src_h.at[pl.ds(base, MB)], si)
            pltpu.sync_copy(dst_h.at[pl.ds(base, MB)], di)
            pltpu.sync_copy(w_h.at[pl.ds(base, MB)], wr)
            pltpu.async_copy(tables.at[si], rw, gs)

        def compute(buf):
            si, di, wr, rw, gs, ss = buf
            pltpu.make_async_copy(tables.at[si], rw, gs).wait()

            def edge(e2, _):
                for u in range(2):
                    e = e2 * 2 + u
                    wv = wr[e]
                    for k in range(nv):
                        rw[e, pl.ds(k * 16, 16)] = (
                            rw[e, pl.ds(k * 16, 16)] * wv)
                return 0
            lax.fori_loop(0, MB // 2, edge, 0)
            pltpu.async_copy(rw, accs.at[di], ss, add=True)

        def swait(buf):
            _, di, _, rw, _, ss = buf
            pltpu.make_async_copy(rw, accs.at[di], ss).wait()

        def do_chunk(j, _):
            gc = cid * half + j

            @pl.when(gc < nchunks)
            def _():
                pltpu.sync_copy(y_h.at[gc, pl.ds(r0, RPT)],
                                tables.at[pl.ds(r0, RPT)])
                pltpu.sync_copy(z_h.at[pl.ds(r0, RPT)],
                                accs.at[pl.ds(r0, RPT)])
                plsc.subcore_barrier()
                fetch(0, bufs[0])

                def pair(p, _):
                    @pl.when(p > 0)
                    def _():
                        swait(bufs[1])
                    fetch(2 * p + 1, bufs[1])
                    compute(bufs[0])

                    @pl.when(p < npair - 1)
                    def _():
                        swait(bufs[0])
                        fetch(2 * p + 2, bufs[0])
                    compute(bufs[1])
                    return 0
                lax.fori_loop(0, npair, pair, 0)
                swait(bufs[0])
                swait(bufs[1])
                plsc.subcore_barrier()
                pltpu.sync_copy(accs.at[pl.ds(r0, RPT)],
                                acc_o.at[gc, pl.ds(r0, RPT)])
                plsc.subcore_barrier()
            return 0
        lax.fori_loop(0, half, do_chunk, 0)

    return body(y_cm, w, src, dst, zeros)


def _il_w(wmat, nch, nf):
    din = wmat.shape[0]
    return wmat.reshape(din, 8, nch, nf).transpose(2, 0, 3, 1).reshape(
        nch, din, nf * 8)


def _il_rows(wmat, nch, nf):
    return wmat.reshape(8, nch, nf, -1).transpose(1, 2, 0, 3).reshape(
        wmat.shape[0], -1)


def _il_att(a, nch, nf):
    heads, f = a.shape
    eye = jnp.eye(heads, dtype=F32)
    m = (eye[:, None, :] * a[:, :, None]).reshape(heads * f, heads)
    m = m.reshape(8, nch, nf, 8).transpose(1, 2, 0, 3).reshape(nch, nf * 8, 8)
    return jnp.concatenate([m, m], axis=2)


def _il_bias(b, nch, nf):
    return b.reshape(8, nch, nf).transpose(1, 2, 0).reshape(nch, 1, nf * 8)


def kernel(x, edge_index, W1, a_src1, a_dst1, b1, W2, a_src2, a_dst2, b2,
           W3, a_src3, a_dst3, b3):
    src = edge_index[0]
    dst = edge_index[1]

    w1cm = _il_w(W1, 16, 8)
    as1 = _il_att(a_src1, 16, 8)
    ad1 = _il_att(a_dst1, 16, 8)
    w2cm = _il_w(_il_rows(W2, 16, 8), 8, 8)
    as2 = _il_att(a_src2, 8, 8)
    ad2 = _il_att(a_dst2, 8, 8)
    w3p = jnp.pad(W3, ((0, 0), (0, 13)))
    w3cm = _il_rows(w3p, 8, 8)[None]
    as3 = jnp.broadcast_to(
        jnp.pad(a_src3[0], (0, 13)).reshape(1, 16, 1), (1, 16, 16))
    ad3 = jnp.broadcast_to(
        jnp.pad(a_dst3[0], (0, 13)).reshape(1, 16, 1), (1, 16, 16))
    b1cm = _il_bias(b1, 16, 8)
    b2cm = _il_bias(b2, 8, 8)
    b3r = jnp.concatenate([b3, jnp.full((13,), -1e30, F32)]).reshape(1, 16)
    zeros16 = jnp.zeros((NP, 16), F32)
    zeros64 = jnp.zeros((NP, 64), F32)
    xp = jnp.pad(x, ((0, NP - N), (0, 0)))

    y1, s1, d1, c1 = _tc_head(xp, w1cm, as1, ad1)
    den1, w1e = _sc_denom(s1, d1, c1, src, dst, zeros16)
    acc1 = _sc_msg(y1, w1e, src, dst, zeros64, 64, 16)

    y2, s2, d2, c2 = _tc_mid(acc1, den1, b1cm, w2cm, as2, ad2)
    den2, w2e = _sc_denom(s2, d2, c2, src, dst, zeros16)
    acc2 = _sc_msg(y2, w2e, src, dst, zeros64, 64, 8)

    y3, s3, d3, c3 = _tc_mid(acc2, den2, b2cm, w3cm, as3, ad3)
    den3, w3e = _sc_denom(s3, d3, c3, src, dst, zeros16)
    acc3 = _sc_msg(y3, w3e, src, dst, zeros16, 16, 1)

    out16 = _tc_final(acc3, den3, b3r)
    return out16[:N, :3]
